# baseline (speedup 1.0000x reference)
import jax
import jax.numpy as jnp
from jax import lax
from jax.experimental import pallas as pl
from jax.experimental.pallas import tpu as pltpu
from jax.experimental.pallas import tpu_sc as plsc

D_IN_ = 768
D_HID_ = 3072
D_OUT_ = 768
E_ = 16
N_ = 128
NSPLIT_ = 8
C_IN_ = D_IN_ // NSPLIT_
C_HID_ = D_HID_ // NSPLIT_


def _logits_kernel(xf_ref, wr_ref, br_ref, out_ref):
    out_ref[...] = (jnp.dot(xf_ref[...], wr_ref[...],
                            preferred_element_type=jnp.float32)
                    + br_ref[...])


def _route_sc_kernel(logits_ref, comb_ref, buf_ref, out_ref):
    info = plsc.get_sparse_core_info()
    nc = info.num_cores
    wid = lax.axis_index("s") * nc + lax.axis_index("c")

    @pl.when(wid < N_ // 16)
    def _work():
        pltpu.sync_copy(logits_ref.at[:, wid], buf_ref)
        v = [buf_ref[e, :] for e in range(E_)]
        m1 = v[0]
        for e in range(1, E_):
            m1 = jnp.maximum(m1, v[e])
        i1 = jnp.full((E_,), jnp.float32(E_))
        for e in range(E_ - 1, -1, -1):
            i1 = jnp.where(v[e] == m1, jnp.float32(e), i1)
        neg_inf = jnp.float32(-jnp.inf)
        m2 = jnp.where(i1 == 0.0, neg_inf, v[0])
        for e in range(1, E_):
            m2 = jnp.maximum(m2, jnp.where(i1 == jnp.float32(e), neg_inf, v[e]))
        i2 = jnp.full((E_,), jnp.float32(E_))
        for e in range(E_ - 1, -1, -1):
            i2 = jnp.where((v[e] == m2) & (i1 != jnp.float32(e)),
                           jnp.float32(e), i2)
        p1 = 1.0 / (1.0 + jnp.exp(m2 - m1))
        p2 = 1.0 - p1
        for e in range(E_):
            ef = jnp.float32(e)
            out_ref[e, :] = (jnp.where(i1 == ef, p1, 0.0)
                             + jnp.where(i2 == ef, p2, 0.0))
        pltpu.sync_copy(out_ref, comb_ref.at[:, wid])


def _moe_kernel(*refs):
    (xf_ref, comb_ref), rest = refs[:2], refs[2:]
    w1_refs = rest[:NSPLIT_]
    b1_ref = rest[NSPLIT_]
    w2_refs = rest[NSPLIT_ + 1:2 * NSPLIT_ + 1]
    b2_ref = rest[2 * NSPLIT_ + 1]
    out_ref = rest[2 * NSPLIT_ + 2]
    e = pl.program_id(0)

    xf = xf_ref[...]
    h = sum(jnp.dot(xf[:, i * C_IN_:(i + 1) * C_IN_], w1_refs[i][0],
                    preferred_element_type=jnp.float32)
            for i in range(NSPLIT_))
    h = jnp.maximum(h + b1_ref[0], 0.0)
    y = sum(jnp.dot(h[:, i * C_HID_:(i + 1) * C_HID_], w2_refs[i][0],
                    preferred_element_type=jnp.float32)
            for i in range(NSPLIT_))
    y = y + b2_ref[0]

    lane = jax.lax.broadcasted_iota(jnp.int32, (N_, E_), 1)
    gate = jnp.sum(jnp.where(lane == e, comb_ref[...], 0.0),
                   axis=1, keepdims=True)
    contrib = gate * y

    @pl.when(e == 0)
    def _init():
        out_ref[...] = contrib

    @pl.when(e != 0)
    def _acc():
        out_ref[...] += contrib


@jax.jit
def kernel(x, Wr, br, W1, b1, W2, b2):
    Bsz, Ssz, d = x.shape
    xf = x.reshape(-1, d)
    n = xf.shape[0]

    logits = pl.pallas_call(
        _logits_kernel,
        in_specs=[
            pl.BlockSpec(memory_space=pltpu.MemorySpace.VMEM),
            pl.BlockSpec(memory_space=pltpu.MemorySpace.VMEM),
            pl.BlockSpec(memory_space=pltpu.MemorySpace.VMEM),
        ],
        out_specs=pl.BlockSpec(memory_space=pltpu.MemorySpace.VMEM),
        out_shape=jax.ShapeDtypeStruct((n, E_), jnp.float32),
    )(xf, Wr, br.reshape(1, E_))

    ngrp = n // 16
    route = pl.kernel(
        _route_sc_kernel,
        mesh=plsc.VectorSubcoreMesh(core_axis_name="c", subcore_axis_name="s"),
        out_type=jax.ShapeDtypeStruct((E_, ngrp, 16), jnp.float32),
        scratch_types=[
            pltpu.VMEM((E_, 16), jnp.float32),
            pltpu.VMEM((E_, 16), jnp.float32),
        ],
    )
    comb_t = route(logits.T.reshape(E_, ngrp, 16))
    comb = comb_t.reshape(E_, n).T

    w1_specs = [pl.BlockSpec((1, C_IN_, D_HID_), lambda e, i=i: (e, i, 0))
                for i in range(NSPLIT_)]
    w2_specs = [pl.BlockSpec((1, C_HID_, D_OUT_), lambda e, i=i: (e, i, 0))
                for i in range(NSPLIT_)]
    out = pl.pallas_call(
        _moe_kernel,
        grid=(E_,),
        in_specs=[
            pl.BlockSpec((n, D_IN_), lambda e: (0, 0)),
            pl.BlockSpec((n, E_), lambda e: (0, 0)),
        ] + w1_specs + [
            pl.BlockSpec((1, 1, D_HID_), lambda e: (e, 0, 0)),
        ] + w2_specs + [
            pl.BlockSpec((1, 1, D_OUT_), lambda e: (e, 0, 0)),
        ],
        out_specs=pl.BlockSpec((n, D_OUT_), lambda e: (0, 0)),
        out_shape=jax.ShapeDtypeStruct((n, D_OUT_), jnp.float32),
    )(xf, comb, *([W1] * NSPLIT_), b1.reshape(E_, 1, D_HID_),
      *([W2] * NSPLIT_), b2.reshape(E_, 1, D_OUT_))
    return out.reshape(Bsz, Ssz, D_OUT_)


# speedup vs baseline: 1.1945x; 1.1945x over previous
import jax
import jax.numpy as jnp
from jax.experimental import pallas as pl
from jax.experimental.pallas import tpu as pltpu

D_IN_ = 768
D_HID_ = 3072
D_OUT_ = 768
E_ = 16
NSPLIT_ = 8
C_IN_ = D_IN_ // NSPLIT_
C_HID_ = D_HID_ // NSPLIT_


NW2_ = 4
W2C_ = D_HID_ // NW2_


def _moe_kernel(*refs):
    (xf_ref, wr_ref, br_ref), rest = refs[:3], refs[3:]
    w1_refs = rest[:NSPLIT_]
    b1_ref = rest[NSPLIT_]
    w2_hbm = rest[NSPLIT_ + 1]
    b2_ref = rest[NSPLIT_ + 2]
    out_ref = rest[NSPLIT_ + 3]
    route_ref = rest[NSPLIT_ + 4]
    w2buf = rest[NSPLIT_ + 5]
    w2sem = rest[NSPLIT_ + 6]
    e = pl.program_id(0)

    def w2_copies(ei):
        slot = jax.lax.rem(ei, 2)
        return [pltpu.make_async_copy(
            w2_hbm.at[ei, pl.ds(c * W2C_, W2C_), :],
            w2buf.at[slot, pl.ds(c * W2C_, W2C_), :],
            w2sem.at[slot, c]) for c in range(NW2_)]

    @pl.when(e == 0)
    def _prime_w2():
        for cp in w2_copies(0):
            cp.start()

    @pl.when(e + 1 < E_)
    def _next_w2():
        for cp in w2_copies(e + 1):
            cp.start()

    @pl.when(e == 0)
    def _compute_routing():
        logits = jnp.dot(xf_ref[...], wr_ref[...],
                         preferred_element_type=jnp.float32)
        logits = logits + br_ref[...]
        n, ecnt = logits.shape
        lane = jax.lax.broadcasted_iota(jnp.int32, (n, ecnt), 1)
        neg_inf = jnp.float32(-jnp.inf)
        m1 = jnp.max(logits, axis=1, keepdims=True)
        i1 = jnp.min(jnp.where(logits == m1, lane, ecnt), axis=1, keepdims=True)
        masked = jnp.where(lane == i1, neg_inf, logits)
        m2 = jnp.max(masked, axis=1, keepdims=True)
        i2 = jnp.min(jnp.where(masked == m2, lane, ecnt), axis=1, keepdims=True)
        p1 = 1.0 / (1.0 + jnp.exp(m2 - m1))
        route_ref[:, 0:1] = i1.astype(jnp.float32)
        route_ref[:, 1:2] = i2.astype(jnp.float32)
        route_ref[:, 2:3] = p1
        route_ref[:, 3:4] = 1.0 - p1

    xf = xf_ref[...]
    h = sum(jnp.dot(xf[:, i * C_IN_:(i + 1) * C_IN_], w1_refs[i][0],
                    preferred_element_type=jnp.float32)
            for i in range(NSPLIT_))
    h = jnp.maximum(h + b1_ref[0], 0.0)
    slot = jax.lax.rem(e, 2)
    y = None
    for c, cp in enumerate(w2_copies(e)):
        cp.wait()
        part = jnp.dot(h[:, c * W2C_:(c + 1) * W2C_],
                       w2buf[slot, pl.ds(c * W2C_, W2C_), :],
                       preferred_element_type=jnp.float32)
        y = part if y is None else y + part
    y = y + b2_ref[0]

    ef = e.astype(jnp.float32)
    gate = (jnp.where(route_ref[:, 0:1] == ef, route_ref[:, 2:3], 0.0)
            + jnp.where(route_ref[:, 1:2] == ef, route_ref[:, 3:4], 0.0))
    contrib = gate * y

    @pl.when(e == 0)
    def _init():
        out_ref[...] = contrib

    @pl.when(e != 0)
    def _acc():
        out_ref[...] += contrib


@jax.jit
def kernel(x, Wr, br, W1, b1, W2, b2):
    Bsz, Ssz, d = x.shape
    xf = x.reshape(-1, d)
    n = xf.shape[0]
    w1_specs = [pl.BlockSpec((1, C_IN_, D_HID_), lambda e, i=i: (e, i, 0))
                for i in range(NSPLIT_)]
    out = pl.pallas_call(
        _moe_kernel,
        grid=(E_,),
        in_specs=[
            pl.BlockSpec((n, D_IN_), lambda e: (0, 0)),
            pl.BlockSpec((D_IN_, E_), lambda e: (0, 0)),
            pl.BlockSpec((1, E_), lambda e: (0, 0)),
        ] + w1_specs + [
            pl.BlockSpec((1, 1, D_HID_), lambda e: (e, 0, 0)),
            pl.BlockSpec(memory_space=pltpu.MemorySpace.HBM),
            pl.BlockSpec((1, 1, D_OUT_), lambda e: (e, 0, 0)),
        ],
        out_specs=pl.BlockSpec((n, D_OUT_), lambda e: (0, 0)),
        out_shape=jax.ShapeDtypeStruct((n, D_OUT_), jnp.float32),
        scratch_shapes=[
            pltpu.VMEM((n, 8), jnp.float32),
            pltpu.VMEM((2, D_HID_, D_OUT_), jnp.float32),
            pltpu.SemaphoreType.DMA((2, NW2_)),
        ],
    )(xf, Wr, br.reshape(1, E_), *([W1] * NSPLIT_),
      b1.reshape(E_, 1, D_HID_), W2, b2.reshape(E_, 1, D_OUT_))
    return out.reshape(Bsz, Ssz, D_OUT_)
